# Initial kernel scaffold; baseline (speedup 1.0000x reference)
import jax
import jax.numpy as jnp
from jax.experimental import pallas as pl


def kernel(atomic_numbers, pos, edge_index, batch, params):
    raise NotImplementedError("write your pallas kernel here")


# speedup vs baseline: 10.6869x; 10.6869x over previous
import functools

import jax
import jax.numpy as jnp
from jax import lax
from jax.experimental import pallas as pl
from jax.experimental.pallas import tpu as pltpu
from jax.experimental.pallas import tpu_sc as plsc

N, E, L, DH, HEADS, HD, B = 10000, 160000, 6, 256, 8, 32, 64
BLK = 512
NBLK = 313
E_PAD = NBLK * BLK
NPB = 21
N_PAD = NPB * BLK
WIN = 528
F32 = jnp.float32


def _ln(x, g, b):
    mu = jnp.mean(x, axis=-1, keepdims=True)
    var = jnp.mean((x - mu) ** 2, axis=-1, keepdims=True)
    return (x - mu) * jax.lax.rsqrt(var + 1e-5) * g + b


def _head_selector():
    c = lax.broadcasted_iota(jnp.int32, (DH, HEADS), 0)
    h = lax.broadcasted_iota(jnp.int32, (DH, HEADS), 1)
    return (c // HD == h).astype(F32)


def _emb_body(an_ref, tab_ref, w_ref, b_ref, g_ref, bb_ref, o_ref):
    an = an_ref[...]
    oh = (an == lax.broadcasted_iota(jnp.int32, (BLK, 119), 1)).astype(F32)
    t = jnp.dot(oh, tab_ref[...], preferred_element_type=F32)
    y = jnp.dot(t, w_ref[...], preferred_element_type=F32) + b_ref[...]
    o_ref[...] = jax.nn.silu(_ln(y, g_ref[...], bb_ref[...]))


def _edgeattr_body(ps_ref, pd_ref, c_ref, w_ref, o_ref):
    diff = ps_ref[...] - pd_ref[...]
    d2 = jnp.sum(diff * diff, axis=1, keepdims=True)
    d = jnp.sqrt(d2)
    gamma = 1.0 / (2.0 * w_ref[...] ** 2)
    rbf = jnp.exp(-gamma * (d - c_ref[...]) ** 2)
    cut = 0.5 * (jnp.cos(jnp.pi * d / 10.0) + 1.0) * (d < 10.0).astype(F32)
    o_ref[...] = rbf * cut


def _dense1_body(x_ref, g_ref, b_ref, wq_ref, bq_ref, wk_ref, bk_ref,
                 wv_ref, bv_ref, ws_ref, bs_ref,
                 q_ref, k_ref, v_ref, s_ref):
    h = _ln(x_ref[...], g_ref[...], b_ref[...])
    q_ref[...] = jnp.dot(h, wq_ref[...], preferred_element_type=F32) + bq_ref[...]
    k_ref[...] = jnp.dot(h, wk_ref[...], preferred_element_type=F32) + bk_ref[...]
    v_ref[...] = jnp.dot(h, wv_ref[...], preferred_element_type=F32) + bv_ref[...]
    s_ref[...] = jnp.dot(h, ws_ref[...], preferred_element_type=F32) + bs_ref[...]


def _logits_body(qe_ref, ke_ref, ea_ref, we_ref, a_ref, gmax_ref):
    i = pl.program_id(0)
    e_blk = jnp.dot(ea_ref[...], we_ref[...], preferred_element_type=F32)
    prod = qe_ref[...] * (ke_ref[...] + e_blk)
    a = jnp.dot(prod, _head_selector(), preferred_element_type=F32)
    a = a * (1.0 / (HD ** 0.5))
    gi = i * BLK + lax.broadcasted_iota(jnp.int32, (BLK, HEADS), 0)
    a = jnp.where(gi < E, a, -1e30)
    a_ref[...] = a

    @pl.when(i == 0)
    def _():
        gmax_ref[...] = jnp.full((1, HEADS), -1e30, F32)

    gmax_ref[...] = jnp.maximum(gmax_ref[...], jnp.max(a, axis=0, keepdims=True))


def _scatter_body(a_ref, ve_ref, ea_ref, we_ref, gmax_ref, dst_ref,
                  msg_ref, den_ref):
    i = pl.program_id(0)

    @pl.when(i == 0)
    def _():
        msg_ref[...] = jnp.zeros((N_PAD, DH), F32)
        den_ref[...] = jnp.zeros((N_PAD, HEADS), F32)

    dst = dst_ref[...]
    start = (dst_ref[0, 0] // 8) * 8
    li = dst - start
    oh = (li == lax.broadcasted_iota(jnp.int32, (BLK, WIN), 1)).astype(F32)
    w = jnp.exp(a_ref[...] - gmax_ref[...])
    sel = _head_selector()
    e_blk = jnp.dot(ea_ref[...], we_ref[...], preferred_element_type=F32)
    vpe = ve_ref[...] + e_blk
    wb = jnp.dot(w, sel.T, preferred_element_type=F32)
    acc = jnp.dot(oh.T, vpe * wb, preferred_element_type=F32)
    dacc = jnp.dot(oh.T, w, preferred_element_type=F32)
    msg_ref[pl.ds(start, WIN), :] = msg_ref[pl.ds(start, WIN), :] + acc
    den_ref[pl.ds(start, WIN), :] = den_ref[pl.ds(start, WIN), :] + dacc


def _combine_body(x_ref, msg_ref, den_ref, sk_ref, al_ref, g_ref, b_ref,
                  w1_ref, b1_ref, w2_ref, b2_ref, o_ref):
    sel = _head_selector()
    dexp = jnp.dot(den_ref[...], sel.T, preferred_element_type=F32)
    attn = msg_ref[...] / (dexp + 1e-16) + sk_ref[...]
    x1 = x_ref[...] + al_ref[0, 0] * attn
    h2 = _ln(x1, g_ref[...], b_ref[...])
    f = jnp.dot(jax.nn.silu(jnp.dot(h2, w1_ref[...], preferred_element_type=F32) + b1_ref[...]),
                w2_ref[...], preferred_element_type=F32) + b2_ref[...]
    o_ref[...] = x1 + f


def _head_body(x4_ref, x5_ref, x6_ref, bt_ref, ew1_ref, eb1_ref, ew2_ref,
               eb2_ref, fw1_ref, fb1_ref, fw2_ref, fb2_ref,
               xm_ref, en_ref, fo_ref):
    i = pl.program_id(0)
    xm = (x4_ref[...] + x5_ref[...] + x6_ref[...]) * (1.0 / 3.0)
    xm_ref[...] = xm
    ae = jnp.dot(jax.nn.silu(jnp.dot(xm, ew1_ref[...], preferred_element_type=F32) + eb1_ref[...]),
                 ew2_ref[...], preferred_element_type=F32) + eb2_ref[...]
    gi = i * BLK + lax.broadcasted_iota(jnp.int32, (BLK, 8), 0)
    ae = jnp.where(gi < N, ae, 0.0)
    bo = (bt_ref[...] == lax.broadcasted_iota(jnp.int32, (BLK, B), 1)).astype(F32)

    @pl.when(i == 0)
    def _():
        en_ref[...] = jnp.zeros((B, 8), F32)

    en_ref[...] = en_ref[...] + jnp.dot(bo.T, ae, preferred_element_type=F32)
    fo_ref[...] = jnp.dot(jax.nn.silu(jnp.dot(xm, fw1_ref[...], preferred_element_type=F32) + fb1_ref[...]),
                          fw2_ref[...], preferred_element_type=F32) + fb2_ref[...]


def _spec(shape, im):
    return pl.BlockSpec(shape, im)


def _w(shape):
    nd = len(shape)
    return pl.BlockSpec(shape, lambda i: (0,) * nd)


def _emb_call(an2, tab, w, b, g, bb):
    return pl.pallas_call(
        _emb_body,
        grid=(NPB,),
        in_specs=[_spec((BLK, 1), lambda i: (i, 0)), _w(tab.shape), _w(w.shape),
                  _w(b.shape), _w(g.shape), _w(bb.shape)],
        out_specs=_spec((BLK, DH), lambda i: (i, 0)),
        out_shape=jax.ShapeDtypeStruct((N_PAD, DH), F32),
    )(an2, tab, w, b, g, bb)


def _edgeattr_call(ps, pd, c, w):
    return pl.pallas_call(
        _edgeattr_body,
        grid=(NBLK,),
        in_specs=[_spec((BLK, 128), lambda i: (i, 0)), _spec((BLK, 128), lambda i: (i, 0)),
                  _w(c.shape), _w(w.shape)],
        out_specs=_spec((BLK, DH), lambda i: (i, 0)),
        out_shape=jax.ShapeDtypeStruct((E_PAD, DH), F32),
    )(ps, pd, c, w)


def _dense1_call(x, g, b, wq, bq, wk, bk, wv, bv, ws, bs):
    eb = _spec((BLK, DH), lambda i: (i, 0))
    return pl.pallas_call(
        _dense1_body,
        grid=(NPB,),
        in_specs=[eb] + [_w(a.shape) for a in (g, b, wq, bq, wk, bk, wv, bv, ws, bs)],
        out_specs=[eb, eb, eb, eb],
        out_shape=[jax.ShapeDtypeStruct((N_PAD, DH), F32)] * 4,
    )(x, g, b, wq, bq, wk, bk, wv, bv, ws, bs)


def _logits_call(qe, ke, ea, we):
    eb = _spec((BLK, DH), lambda i: (i, 0))
    return pl.pallas_call(
        _logits_body,
        grid=(NBLK,),
        in_specs=[eb, eb, eb, _w(we.shape)],
        out_specs=[_spec((BLK, HEADS), lambda i: (i, 0)), _w((1, HEADS))],
        out_shape=[jax.ShapeDtypeStruct((E_PAD, HEADS), F32),
                   jax.ShapeDtypeStruct((1, HEADS), F32)],
    )(qe, ke, ea, we)


def _scatter_call(a, ve, ea, we, gmax, dst2):
    eb = _spec((BLK, DH), lambda i: (i, 0))
    return pl.pallas_call(
        _scatter_body,
        grid=(NBLK,),
        in_specs=[_spec((BLK, HEADS), lambda i: (i, 0)), eb, eb, _w(we.shape),
                  _w((1, HEADS)), _spec((BLK, 1), lambda i: (i, 0))],
        out_specs=[_w((N_PAD, DH)), _w((N_PAD, HEADS))],
        out_shape=[jax.ShapeDtypeStruct((N_PAD, DH), F32),
                   jax.ShapeDtypeStruct((N_PAD, HEADS), F32)],
    )(a, ve, ea, we, gmax, dst2)


def _combine_call(x, msg, den, sk, al, g, b, w1, b1, w2, b2):
    eb = _spec((BLK, DH), lambda i: (i, 0))
    return pl.pallas_call(
        _combine_body,
        grid=(NPB,),
        in_specs=[eb, eb, _spec((BLK, HEADS), lambda i: (i, 0)), eb,
                  _w(al.shape), _w(g.shape), _w(b.shape), _w(w1.shape),
                  _w(b1.shape), _w(w2.shape), _w(b2.shape)],
        out_specs=eb,
        out_shape=jax.ShapeDtypeStruct((N_PAD, DH), F32),
    )(x, msg, den, sk, al, g, b, w1, b1, w2, b2)


def _head_call(x4, x5, x6, bt2, ew1, eb1, ew2, eb2, fw1, fb1, fw2, fb2):
    eb = _spec((BLK, DH), lambda i: (i, 0))
    return pl.pallas_call(
        _head_body,
        grid=(NPB,),
        in_specs=[eb, eb, eb, _spec((BLK, 1), lambda i: (i, 0))]
                 + [_w(a.shape) for a in (ew1, eb1, ew2, eb2, fw1, fb1, fw2, fb2)],
        out_specs=[eb, _w((B, 8)), _spec((BLK, 8), lambda i: (i, 0))],
        out_shape=[jax.ShapeDtypeStruct((N_PAD, DH), F32),
                   jax.ShapeDtypeStruct((B, 8), F32),
                   jax.ShapeDtypeStruct((N_PAD, 8), F32)],
    )(x4, x5, x6, bt2, ew1, eb1, ew2, eb2, fw1, fb1, fw2, fb2)


CHUNK = 16


def _sc_gather2(table, idx_a, idx_b, d):
    info = plsc.get_sparse_core_info()
    nc, ns = info.num_cores, info.num_subcores
    nw = nc * ns
    bpw = E_PAD // nw
    nch = bpw // CHUNK
    mesh = plsc.VectorSubcoreMesh(core_axis_name="c", subcore_axis_name="s")

    @functools.partial(
        pl.kernel, mesh=mesh,
        out_type=[jax.ShapeDtypeStruct((E_PAD, d), F32)] * 2,
        scratch_types=[
            pltpu.VMEM((CHUNK,), jnp.int32), pltpu.VMEM((CHUNK,), jnp.int32),
            pltpu.VMEM((CHUNK, d), F32), pltpu.VMEM((CHUNK, d), F32),
            pltpu.SemaphoreType.DMA, pltpu.SemaphoreType.DMA,
        ],
    )
    def k(tab_h, ia_h, ib_h, oa_h, ob_h, ia_v, ib_v, ra_v, rb_v, sa, sb):
        wid = lax.axis_index("s") * nc + lax.axis_index("c")
        base = wid * bpw

        def body(c, _):
            off = base + c * CHUNK
            pltpu.sync_copy(ia_h.at[pl.ds(off, CHUNK)], ia_v)
            pltpu.sync_copy(ib_h.at[pl.ds(off, CHUNK)], ib_v)
            ca = pltpu.async_copy(tab_h.at[ia_v], ra_v, sa)
            cb = pltpu.async_copy(tab_h.at[ib_v], rb_v, sb)
            ca.wait()
            cb.wait()
            pltpu.sync_copy(ra_v, oa_h.at[pl.ds(off, CHUNK)])
            pltpu.sync_copy(rb_v, ob_h.at[pl.ds(off, CHUNK)])
            return _

        lax.fori_loop(0, nch, body, None)

    return k(table, idx_a, idx_b)


def _sc_gather3(tq, tk, tv, idx_dst, idx_src, d):
    info = plsc.get_sparse_core_info()
    nc, ns = info.num_cores, info.num_subcores
    nw = nc * ns
    bpw = E_PAD // nw
    nch = bpw // CHUNK
    mesh = plsc.VectorSubcoreMesh(core_axis_name="c", subcore_axis_name="s")

    @functools.partial(
        pl.kernel, mesh=mesh,
        out_type=[jax.ShapeDtypeStruct((E_PAD, d), F32)] * 3,
        scratch_types=[
            pltpu.VMEM((CHUNK,), jnp.int32), pltpu.VMEM((CHUNK,), jnp.int32),
            pltpu.VMEM((CHUNK, d), F32), pltpu.VMEM((CHUNK, d), F32),
            pltpu.VMEM((CHUNK, d), F32),
            pltpu.SemaphoreType.DMA, pltpu.SemaphoreType.DMA,
            pltpu.SemaphoreType.DMA,
        ],
    )
    def k(tq_h, tk_h, tv_h, id_h, is_h, oq_h, ok_h, ov_h,
          id_v, is_v, rq_v, rk_v, rv_v, sq, sk, sv):
        wid = lax.axis_index("s") * nc + lax.axis_index("c")
        base = wid * bpw

        def body(c, _):
            off = base + c * CHUNK
            pltpu.sync_copy(id_h.at[pl.ds(off, CHUNK)], id_v)
            pltpu.sync_copy(is_h.at[pl.ds(off, CHUNK)], is_v)
            cq = pltpu.async_copy(tq_h.at[id_v], rq_v, sq)
            ck = pltpu.async_copy(tk_h.at[is_v], rk_v, sk)
            cv = pltpu.async_copy(tv_h.at[is_v], rv_v, sv)
            cq.wait()
            ck.wait()
            cv.wait()
            pltpu.sync_copy(rq_v, oq_h.at[pl.ds(off, CHUNK)])
            pltpu.sync_copy(rk_v, ok_h.at[pl.ds(off, CHUNK)])
            pltpu.sync_copy(rv_v, ov_h.at[pl.ds(off, CHUNK)])
            return _

        lax.fori_loop(0, nch, body, None)

    return k(tq, tk, tv, idx_dst, idx_src)


def kernel(atomic_numbers, pos, edge_index, batch, params):
    p = params
    r2 = lambda a: a.reshape(1, -1).astype(F32)

    src = edge_index[0].astype(jnp.int32)
    dst = edge_index[1].astype(jnp.int32)
    perm = jnp.argsort(dst)
    src_s = src[perm]
    dst_s = dst[perm]
    padn = E_PAD - E
    src_sp = jnp.concatenate([src_s, jnp.zeros((padn,), jnp.int32)])
    dst_gp = jnp.concatenate([dst_s, jnp.zeros((padn,), jnp.int32)])
    dst_wp = jnp.concatenate([dst_s, jnp.full((padn,), dst_s[-1], jnp.int32)])
    dst2 = dst_wp.reshape(E_PAD, 1)
    an2 = jnp.concatenate([atomic_numbers.astype(jnp.int32),
                           jnp.zeros((N_PAD - N,), jnp.int32)]).reshape(N_PAD, 1)
    bt2 = jnp.concatenate([batch.astype(jnp.int32),
                           jnp.full((N_PAD - N,), B - 1, jnp.int32)]).reshape(N_PAD, 1)
    posp = jnp.pad(pos.astype(F32), ((0, 0), (0, 125)))

    tab = jnp.concatenate([p['elem_emb'], p['radius_emb'], p['en_emb'], p['ie_emb']],
                          axis=1).astype(F32)
    x = _emb_call(an2, tab, p['proj_W'].astype(F32), r2(p['proj_b']),
                  r2(p['proj_ln_g']), r2(p['proj_ln_b']))

    ps, pd = _sc_gather2(posp, src_sp, dst_gp, 128)
    ea = _edgeattr_call(ps, pd, r2(p['rbf_centers']), r2(p['rbf_widths']))

    feats = [x]
    for l in range(L):
        q, k, v, sk = _dense1_call(
            x, r2(p['n1_g'][l]), r2(p['n1_b'][l]),
            p['Wq'][l], r2(p['bq'][l]), p['Wk'][l], r2(p['bk'][l]),
            p['Wv'][l], r2(p['bv'][l]), p['Wskip'][l], r2(p['bskip'][l]))
        qe, ke, ve = _sc_gather3(q, k, v, dst_gp, src_sp, DH)
        we = p['We'][l].astype(F32)
        a, gmax = _logits_call(qe, ke, ea, we)
        msg, den = _scatter_call(a, ve, ea, we, gmax, dst2)
        x = _combine_call(x, msg, den, sk, p['alpha'][l].reshape(1, 1),
                          r2(p['n2_g'][l]), r2(p['n2_b'][l]),
                          p['f_W1'][l], r2(p['f_b1'][l]),
                          p['f_W2'][l], r2(p['f_b2'][l]))
        feats.append(x)

    ew2 = jnp.pad(p['e_W2'].astype(F32), ((0, 0), (0, 7)))
    eb2 = jnp.pad(p['e_b2'].astype(F32).reshape(1, 1), ((0, 0), (0, 7)))
    fw2 = jnp.pad(p['fr_W2'].astype(F32), ((0, 0), (0, 5)))
    fb2 = jnp.pad(p['fr_b2'].astype(F32).reshape(1, 3), ((0, 0), (0, 5)))
    xm, en, fo = _head_call(feats[L - 2], feats[L - 1], feats[L], bt2,
                            p['e_W1'].astype(F32), r2(p['e_b1']), ew2, eb2,
                            p['fr_W1'].astype(F32), r2(p['fr_b1']), fw2, fb2)
    energy = en[:, 0]
    forces = fo[:N, :3]
    return energy, forces, xm[:N]


# speedup vs baseline: 12.2853x; 1.1496x over previous
import functools

import jax
import jax.numpy as jnp
from jax import lax
from jax.experimental import pallas as pl
from jax.experimental.pallas import tpu as pltpu
from jax.experimental.pallas import tpu_sc as plsc

N, E, L, DH, HEADS, HD, B = 10000, 160000, 6, 256, 8, 32, 64
BLK = 512
NBLK = 320
E_PAD = NBLK * BLK
NPB = 21
N_PAD = NPB * BLK
WIN = 528
F32 = jnp.float32


def _ln(x, g, b):
    mu = jnp.mean(x, axis=-1, keepdims=True)
    var = jnp.mean((x - mu) ** 2, axis=-1, keepdims=True)
    return (x - mu) * jax.lax.rsqrt(var + 1e-5) * g + b


def _head_selector():
    c = lax.broadcasted_iota(jnp.int32, (DH, HEADS), 0)
    h = lax.broadcasted_iota(jnp.int32, (DH, HEADS), 1)
    return (c // HD == h).astype(F32)


def _emb_body(an_ref, tab_ref, w_ref, b_ref, g_ref, bb_ref, o_ref):
    an = an_ref[...]
    oh = (an == lax.broadcasted_iota(jnp.int32, (BLK, 119), 1)).astype(F32)
    t = jnp.dot(oh, tab_ref[...], preferred_element_type=F32)
    y = jnp.dot(t, w_ref[...], preferred_element_type=F32) + b_ref[...]
    o_ref[...] = jax.nn.silu(_ln(y, g_ref[...], bb_ref[...]))


def _edgeattr_body(ps_ref, pd_ref, c_ref, w_ref, o_ref):
    diff = ps_ref[...] - pd_ref[...]
    d2 = jnp.sum(diff * diff, axis=1, keepdims=True)
    d = jnp.sqrt(d2)
    gamma = 1.0 / (2.0 * w_ref[...] ** 2)
    rbf = jnp.exp(-gamma * (d - c_ref[...]) ** 2)
    cut = 0.5 * (jnp.cos(jnp.pi * d / 10.0) + 1.0) * (d < 10.0).astype(F32)
    o_ref[...] = rbf * cut


def _dense1_body(x_ref, g_ref, b_ref, wq_ref, bq_ref, wk_ref, bk_ref,
                 wv_ref, bv_ref, ws_ref, bs_ref,
                 q_ref, k_ref, v_ref, s_ref):
    h = _ln(x_ref[...], g_ref[...], b_ref[...])
    q_ref[...] = jnp.dot(h, wq_ref[...], preferred_element_type=F32) + bq_ref[...]
    k_ref[...] = jnp.dot(h, wk_ref[...], preferred_element_type=F32) + bk_ref[...]
    v_ref[...] = jnp.dot(h, wv_ref[...], preferred_element_type=F32) + bv_ref[...]
    s_ref[...] = jnp.dot(h, ws_ref[...], preferred_element_type=F32) + bs_ref[...]


def _logits_body(qe_ref, ke_ref, ea_ref, we_ref, a_ref, gmax_ref):
    i = pl.program_id(0)
    e_blk = jnp.dot(ea_ref[...], we_ref[...], preferred_element_type=F32)
    prod = qe_ref[...] * (ke_ref[...] + e_blk)
    a = jnp.dot(prod, _head_selector(), preferred_element_type=F32)
    a = a * (1.0 / (HD ** 0.5))
    gi = i * BLK + lax.broadcasted_iota(jnp.int32, (BLK, HEADS), 0)
    a = jnp.where(gi < E, a, -1e30)
    a_ref[...] = a

    @pl.when(i == 0)
    def _():
        gmax_ref[...] = jnp.full((1, HEADS), -1e30, F32)

    gmax_ref[...] = jnp.maximum(gmax_ref[...], jnp.max(a, axis=0, keepdims=True))


def _scatter_body(a_ref, ve_ref, ea_ref, we_ref, gmax_ref, dst_ref,
                  msg_ref, den_ref):
    i = pl.program_id(0)

    @pl.when(i == 0)
    def _():
        msg_ref[...] = jnp.zeros((N_PAD, DH), F32)
        den_ref[...] = jnp.zeros((N_PAD, HEADS), F32)

    dst = dst_ref[...]
    start = (dst_ref[0, 0] // 8) * 8
    li = dst - start
    oh = (li == lax.broadcasted_iota(jnp.int32, (BLK, WIN), 1)).astype(F32)
    w = jnp.exp(a_ref[...] - gmax_ref[...])
    sel = _head_selector()
    e_blk = jnp.dot(ea_ref[...], we_ref[...], preferred_element_type=F32)
    vpe = ve_ref[...] + e_blk
    wb = jnp.dot(w, sel.T, preferred_element_type=F32)
    acc = jnp.dot(oh.T, vpe * wb, preferred_element_type=F32)
    dacc = jnp.dot(oh.T, w, preferred_element_type=F32)
    msg_ref[pl.ds(start, WIN), :] = msg_ref[pl.ds(start, WIN), :] + acc
    den_ref[pl.ds(start, WIN), :] = den_ref[pl.ds(start, WIN), :] + dacc


def _combine_body(x_ref, msg_ref, den_ref, sk_ref, al_ref, g_ref, b_ref,
                  w1_ref, b1_ref, w2_ref, b2_ref, o_ref):
    sel = _head_selector()
    dexp = jnp.dot(den_ref[...], sel.T, preferred_element_type=F32)
    attn = msg_ref[...] / (dexp + 1e-16) + sk_ref[...]
    x1 = x_ref[...] + al_ref[0, 0] * attn
    h2 = _ln(x1, g_ref[...], b_ref[...])
    f = jnp.dot(jax.nn.silu(jnp.dot(h2, w1_ref[...], preferred_element_type=F32) + b1_ref[...]),
                w2_ref[...], preferred_element_type=F32) + b2_ref[...]
    o_ref[...] = x1 + f


def _head_body(x4_ref, x5_ref, x6_ref, bt_ref, ew1_ref, eb1_ref, ew2_ref,
               eb2_ref, fw1_ref, fb1_ref, fw2_ref, fb2_ref,
               xm_ref, en_ref, fo_ref):
    i = pl.program_id(0)
    xm = (x4_ref[...] + x5_ref[...] + x6_ref[...]) * (1.0 / 3.0)
    xm_ref[...] = xm
    ae = jnp.dot(jax.nn.silu(jnp.dot(xm, ew1_ref[...], preferred_element_type=F32) + eb1_ref[...]),
                 ew2_ref[...], preferred_element_type=F32) + eb2_ref[...]
    gi = i * BLK + lax.broadcasted_iota(jnp.int32, (BLK, 8), 0)
    ae = jnp.where(gi < N, ae, 0.0)
    bo = (bt_ref[...] == lax.broadcasted_iota(jnp.int32, (BLK, B), 1)).astype(F32)

    @pl.when(i == 0)
    def _():
        en_ref[...] = jnp.zeros((B, 8), F32)

    en_ref[...] = en_ref[...] + jnp.dot(bo.T, ae, preferred_element_type=F32)
    fo_ref[...] = jnp.dot(jax.nn.silu(jnp.dot(xm, fw1_ref[...], preferred_element_type=F32) + fb1_ref[...]),
                          fw2_ref[...], preferred_element_type=F32) + fb2_ref[...]


def _spec(shape, im):
    return pl.BlockSpec(shape, im)


def _w(shape):
    nd = len(shape)
    return pl.BlockSpec(shape, lambda i: (0,) * nd)


def _emb_call(an2, tab, w, b, g, bb):
    return pl.pallas_call(
        _emb_body,
        grid=(NPB,),
        in_specs=[_spec((BLK, 1), lambda i: (i, 0)), _w(tab.shape), _w(w.shape),
                  _w(b.shape), _w(g.shape), _w(bb.shape)],
        out_specs=_spec((BLK, DH), lambda i: (i, 0)),
        out_shape=jax.ShapeDtypeStruct((N_PAD, DH), F32),
    )(an2, tab, w, b, g, bb)


def _edgeattr_call(ps, pd, c, w):
    return pl.pallas_call(
        _edgeattr_body,
        grid=(NBLK,),
        in_specs=[_spec((BLK, 128), lambda i: (i, 0)), _spec((BLK, 128), lambda i: (i, 0)),
                  _w(c.shape), _w(w.shape)],
        out_specs=_spec((BLK, DH), lambda i: (i, 0)),
        out_shape=jax.ShapeDtypeStruct((E_PAD, DH), F32),
    )(ps, pd, c, w)


def _dense1_call(x, g, b, wq, bq, wk, bk, wv, bv, ws, bs):
    eb = _spec((BLK, DH), lambda i: (i, 0))
    return pl.pallas_call(
        _dense1_body,
        grid=(NPB,),
        in_specs=[eb] + [_w(a.shape) for a in (g, b, wq, bq, wk, bk, wv, bv, ws, bs)],
        out_specs=[eb, eb, eb, eb],
        out_shape=[jax.ShapeDtypeStruct((N_PAD, DH), F32)] * 4,
    )(x, g, b, wq, bq, wk, bk, wv, bv, ws, bs)


def _logits_call(qe, ke, ea, we):
    eb = _spec((BLK, DH), lambda i: (i, 0))
    return pl.pallas_call(
        _logits_body,
        grid=(NBLK,),
        in_specs=[eb, eb, eb, _w(we.shape)],
        out_specs=[_spec((BLK, HEADS), lambda i: (i, 0)), _w((1, HEADS))],
        out_shape=[jax.ShapeDtypeStruct((E_PAD, HEADS), F32),
                   jax.ShapeDtypeStruct((1, HEADS), F32)],
    )(qe, ke, ea, we)


def _scatter_call(a, ve, ea, we, gmax, dst2):
    eb = _spec((BLK, DH), lambda i: (i, 0))
    return pl.pallas_call(
        _scatter_body,
        grid=(NBLK,),
        in_specs=[_spec((BLK, HEADS), lambda i: (i, 0)), eb, eb, _w(we.shape),
                  _w((1, HEADS)), _spec((BLK, 1), lambda i: (i, 0))],
        out_specs=[_w((N_PAD, DH)), _w((N_PAD, HEADS))],
        out_shape=[jax.ShapeDtypeStruct((N_PAD, DH), F32),
                   jax.ShapeDtypeStruct((N_PAD, HEADS), F32)],
    )(a, ve, ea, we, gmax, dst2)


def _combine_call(x, msg, den, sk, al, g, b, w1, b1, w2, b2):
    eb = _spec((BLK, DH), lambda i: (i, 0))
    return pl.pallas_call(
        _combine_body,
        grid=(NPB,),
        in_specs=[eb, eb, _spec((BLK, HEADS), lambda i: (i, 0)), eb,
                  _w(al.shape), _w(g.shape), _w(b.shape), _w(w1.shape),
                  _w(b1.shape), _w(w2.shape), _w(b2.shape)],
        out_specs=eb,
        out_shape=jax.ShapeDtypeStruct((N_PAD, DH), F32),
    )(x, msg, den, sk, al, g, b, w1, b1, w2, b2)


def _head_call(x4, x5, x6, bt2, ew1, eb1, ew2, eb2, fw1, fb1, fw2, fb2):
    eb = _spec((BLK, DH), lambda i: (i, 0))
    return pl.pallas_call(
        _head_body,
        grid=(NPB,),
        in_specs=[eb, eb, eb, _spec((BLK, 1), lambda i: (i, 0))]
                 + [_w(a.shape) for a in (ew1, eb1, ew2, eb2, fw1, fb1, fw2, fb2)],
        out_specs=[eb, _w((B, 8)), _spec((BLK, 8), lambda i: (i, 0))],
        out_shape=[jax.ShapeDtypeStruct((N_PAD, DH), F32),
                   jax.ShapeDtypeStruct((B, 8), F32),
                   jax.ShapeDtypeStruct((N_PAD, 8), F32)],
    )(x4, x5, x6, bt2, ew1, eb1, ew2, eb2, fw1, fb1, fw2, fb2)


CHUNK = 64


def _sc_gather2(table, idx_a, idx_b, d):
    info = plsc.get_sparse_core_info()
    nc, ns = info.num_cores, info.num_subcores
    nw = nc * ns
    bpw = E_PAD // nw
    nch = bpw // CHUNK
    mesh = plsc.VectorSubcoreMesh(core_axis_name="c", subcore_axis_name="s")

    @functools.partial(
        pl.kernel, mesh=mesh,
        out_type=[jax.ShapeDtypeStruct((E_PAD, d), F32)] * 2,
        scratch_types=[
            pltpu.VMEM((CHUNK,), jnp.int32), pltpu.VMEM((CHUNK,), jnp.int32),
            pltpu.VMEM((CHUNK, d), F32), pltpu.VMEM((CHUNK, d), F32),
            pltpu.SemaphoreType.DMA, pltpu.SemaphoreType.DMA,
        ],
    )
    def k(tab_h, ia_h, ib_h, oa_h, ob_h, ia_v, ib_v, ra_v, rb_v, sa, sb):
        wid = lax.axis_index("s") * nc + lax.axis_index("c")
        base = wid * bpw

        def body(c, _):
            off = base + c * CHUNK
            pltpu.sync_copy(ia_h.at[pl.ds(off, CHUNK)], ia_v)
            pltpu.sync_copy(ib_h.at[pl.ds(off, CHUNK)], ib_v)
            ca = pltpu.async_copy(tab_h.at[ia_v], ra_v, sa)
            cb = pltpu.async_copy(tab_h.at[ib_v], rb_v, sb)
            ca.wait()
            cb.wait()
            pltpu.sync_copy(ra_v, oa_h.at[pl.ds(off, CHUNK)])
            pltpu.sync_copy(rb_v, ob_h.at[pl.ds(off, CHUNK)])
            return _

        lax.fori_loop(0, nch, body, None)

    return k(table, idx_a, idx_b)


def _sc_gather3(tq, tk, tv, idx_dst, idx_src, d):
    info = plsc.get_sparse_core_info()
    nc, ns = info.num_cores, info.num_subcores
    nw = nc * ns
    bpw = E_PAD // nw
    nch = bpw // CHUNK
    mesh = plsc.VectorSubcoreMesh(core_axis_name="c", subcore_axis_name="s")

    @functools.partial(
        pl.kernel, mesh=mesh,
        out_type=[jax.ShapeDtypeStruct((E_PAD, d), F32)] * 3,
        scratch_types=[
            pltpu.VMEM((CHUNK,), jnp.int32), pltpu.VMEM((CHUNK,), jnp.int32),
            pltpu.VMEM((CHUNK, d), F32), pltpu.VMEM((CHUNK, d), F32),
            pltpu.VMEM((CHUNK, d), F32),
            pltpu.SemaphoreType.DMA, pltpu.SemaphoreType.DMA,
            pltpu.SemaphoreType.DMA,
        ],
    )
    def k(tq_h, tk_h, tv_h, id_h, is_h, oq_h, ok_h, ov_h,
          id_v, is_v, rq_v, rk_v, rv_v, sq, sk, sv):
        wid = lax.axis_index("s") * nc + lax.axis_index("c")
        base = wid * bpw

        def body(c, _):
            off = base + c * CHUNK
            pltpu.sync_copy(id_h.at[pl.ds(off, CHUNK)], id_v)
            pltpu.sync_copy(is_h.at[pl.ds(off, CHUNK)], is_v)
            cq = pltpu.async_copy(tq_h.at[id_v], rq_v, sq)
            ck = pltpu.async_copy(tk_h.at[is_v], rk_v, sk)
            cv = pltpu.async_copy(tv_h.at[is_v], rv_v, sv)
            cq.wait()
            ck.wait()
            cv.wait()
            pltpu.sync_copy(rq_v, oq_h.at[pl.ds(off, CHUNK)])
            pltpu.sync_copy(rk_v, ok_h.at[pl.ds(off, CHUNK)])
            pltpu.sync_copy(rv_v, ov_h.at[pl.ds(off, CHUNK)])
            return _

        lax.fori_loop(0, nch, body, None)

    return k(tq, tk, tv, idx_dst, idx_src)


def kernel(atomic_numbers, pos, edge_index, batch, params):
    p = params
    r2 = lambda a: a.reshape(1, -1).astype(F32)

    src = edge_index[0].astype(jnp.int32)
    dst = edge_index[1].astype(jnp.int32)
    perm = jnp.argsort(dst)
    src_s = src[perm]
    dst_s = dst[perm]
    padn = E_PAD - E
    src_sp = jnp.concatenate([src_s, jnp.zeros((padn,), jnp.int32)])
    dst_gp = jnp.concatenate([dst_s, jnp.zeros((padn,), jnp.int32)])
    dst_wp = jnp.concatenate([dst_s, jnp.full((padn,), dst_s[-1], jnp.int32)])
    dst2 = dst_wp.reshape(E_PAD, 1)
    an2 = jnp.concatenate([atomic_numbers.astype(jnp.int32),
                           jnp.zeros((N_PAD - N,), jnp.int32)]).reshape(N_PAD, 1)
    bt2 = jnp.concatenate([batch.astype(jnp.int32),
                           jnp.full((N_PAD - N,), B - 1, jnp.int32)]).reshape(N_PAD, 1)
    posp = jnp.pad(pos.astype(F32), ((0, 0), (0, 125)))

    tab = jnp.concatenate([p['elem_emb'], p['radius_emb'], p['en_emb'], p['ie_emb']],
                          axis=1).astype(F32)
    x = _emb_call(an2, tab, p['proj_W'].astype(F32), r2(p['proj_b']),
                  r2(p['proj_ln_g']), r2(p['proj_ln_b']))

    ps, pd = _sc_gather2(posp, src_sp, dst_gp, 128)
    ea = _edgeattr_call(ps, pd, r2(p['rbf_centers']), r2(p['rbf_widths']))

    feats = [x]
    for l in range(L):
        q, k, v, sk = _dense1_call(
            x, r2(p['n1_g'][l]), r2(p['n1_b'][l]),
            p['Wq'][l], r2(p['bq'][l]), p['Wk'][l], r2(p['bk'][l]),
            p['Wv'][l], r2(p['bv'][l]), p['Wskip'][l], r2(p['bskip'][l]))
        qe, ke, ve = _sc_gather3(q, k, v, dst_gp, src_sp, DH)
        we = p['We'][l].astype(F32)
        a, gmax = _logits_call(qe, ke, ea, we)
        msg, den = _scatter_call(a, ve, ea, we, gmax, dst2)
        x = _combine_call(x, msg, den, sk, p['alpha'][l].reshape(1, 1),
                          r2(p['n2_g'][l]), r2(p['n2_b'][l]),
                          p['f_W1'][l], r2(p['f_b1'][l]),
                          p['f_W2'][l], r2(p['f_b2'][l]))
        feats.append(x)

    ew2 = jnp.pad(p['e_W2'].astype(F32), ((0, 0), (0, 7)))
    eb2 = jnp.pad(p['e_b2'].astype(F32).reshape(1, 1), ((0, 0), (0, 7)))
    fw2 = jnp.pad(p['fr_W2'].astype(F32), ((0, 0), (0, 5)))
    fb2 = jnp.pad(p['fr_b2'].astype(F32).reshape(1, 3), ((0, 0), (0, 5)))
    xm, en, fo = _head_call(feats[L - 2], feats[L - 1], feats[L], bt2,
                            p['e_W1'].astype(F32), r2(p['e_b1']), ew2, eb2,
                            p['fr_W1'].astype(F32), r2(p['fr_b1']), fw2, fb2)
    energy = en[:, 0]
    forces = fo[:N, :3]
    return energy, forces, xm[:N]


# speedup vs baseline: 13.5321x; 1.1015x over previous
import functools

import jax
import jax.numpy as jnp
from jax import lax
from jax.experimental import pallas as pl
from jax.experimental.pallas import tpu as pltpu
from jax.experimental.pallas import tpu_sc as plsc

N, E, L, DH, HEADS, HD, B = 10000, 160000, 6, 256, 8, 32, 64
BLK = 512
NBLK = 320
E_PAD = NBLK * BLK
NPB = 21
N_PAD = NPB * BLK
WIN = 528
F32 = jnp.float32


def _ln(x, g, b):
    mu = jnp.mean(x, axis=-1, keepdims=True)
    var = jnp.mean((x - mu) ** 2, axis=-1, keepdims=True)
    return (x - mu) * jax.lax.rsqrt(var + 1e-5) * g + b


def _head_selector():
    c = lax.broadcasted_iota(jnp.int32, (DH, HEADS), 0)
    h = lax.broadcasted_iota(jnp.int32, (DH, HEADS), 1)
    return (c // HD == h).astype(F32)


def _emb_body(an_ref, tab_ref, w_ref, b_ref, g_ref, bb_ref, o_ref):
    an = an_ref[...]
    oh = (an == lax.broadcasted_iota(jnp.int32, (BLK, 119), 1)).astype(F32)
    t = jnp.dot(oh, tab_ref[...], preferred_element_type=F32)
    y = jnp.dot(t, w_ref[...], preferred_element_type=F32) + b_ref[...]
    o_ref[...] = jax.nn.silu(_ln(y, g_ref[...], bb_ref[...]))


def _edgeattr_body(ps_ref, pd_ref, c_ref, w_ref, o_ref):
    diff = ps_ref[...] - pd_ref[...]
    d2 = jnp.sum(diff * diff, axis=1, keepdims=True)
    d = jnp.sqrt(d2)
    gamma = 1.0 / (2.0 * w_ref[...] ** 2)
    rbf = jnp.exp(-gamma * (d - c_ref[...]) ** 2)
    cut = 0.5 * (jnp.cos(jnp.pi * d / 10.0) + 1.0) * (d < 10.0).astype(F32)
    o_ref[...] = rbf * cut


def _dense1_body(x_ref, g_ref, b_ref, wq_ref, bq_ref, wk_ref, bk_ref,
                 wv_ref, bv_ref, ws_ref, bs_ref,
                 q_ref, kv_ref, s_ref):
    h = _ln(x_ref[...], g_ref[...], b_ref[...])
    q_ref[...] = jnp.dot(h, wq_ref[...], preferred_element_type=F32) + bq_ref[...]
    kv_ref[:, :DH] = jnp.dot(h, wk_ref[...], preferred_element_type=F32) + bk_ref[...]
    kv_ref[:, DH:] = jnp.dot(h, wv_ref[...], preferred_element_type=F32) + bv_ref[...]
    s_ref[...] = jnp.dot(h, ws_ref[...], preferred_element_type=F32) + bs_ref[...]


def _logits_body(q_ref, kve_ref, ea_ref, we_ref, dst_ref, a_ref, gmax_ref):
    i = pl.program_id(0)
    dst = dst_ref[...]
    start = (dst_ref[0, 0] // 8) * 8
    li = dst - start
    oh = (li == lax.broadcasted_iota(jnp.int32, (BLK, WIN), 1)).astype(F32)
    qe = jnp.dot(oh, q_ref[pl.ds(start, WIN), :], preferred_element_type=F32)
    e_blk = jnp.dot(ea_ref[...], we_ref[...], preferred_element_type=F32)
    prod = qe * (kve_ref[:, :DH] + e_blk)
    a = jnp.dot(prod, _head_selector(), preferred_element_type=F32)
    a = a * (1.0 / (HD ** 0.5))
    gi = i * BLK + lax.broadcasted_iota(jnp.int32, (BLK, HEADS), 0)
    a = jnp.where(gi < E, a, -1e30)
    a_ref[...] = a

    @pl.when(i == 0)
    def _():
        gmax_ref[...] = jnp.full((1, HEADS), -1e30, F32)

    gmax_ref[...] = jnp.maximum(gmax_ref[...], jnp.max(a, axis=0, keepdims=True))


def _scatter_body(a_ref, kve_ref, ea_ref, we_ref, gmax_ref, dst_ref,
                  msg_ref, den_ref):
    i = pl.program_id(0)

    @pl.when(i == 0)
    def _():
        msg_ref[...] = jnp.zeros((N_PAD, DH), F32)
        den_ref[...] = jnp.zeros((N_PAD, HEADS), F32)

    dst = dst_ref[...]
    start = (dst_ref[0, 0] // 8) * 8
    li = dst - start
    oh = (li == lax.broadcasted_iota(jnp.int32, (BLK, WIN), 1)).astype(F32)
    w = jnp.exp(a_ref[...] - gmax_ref[...])
    sel = _head_selector()
    e_blk = jnp.dot(ea_ref[...], we_ref[...], preferred_element_type=F32)
    vpe = kve_ref[:, DH:] + e_blk
    wb = jnp.dot(w, sel.T, preferred_element_type=F32)
    acc = jnp.dot(oh.T, vpe * wb, preferred_element_type=F32)
    dacc = jnp.dot(oh.T, w, preferred_element_type=F32)
    msg_ref[pl.ds(start, WIN), :] = msg_ref[pl.ds(start, WIN), :] + acc
    den_ref[pl.ds(start, WIN), :] = den_ref[pl.ds(start, WIN), :] + dacc


def _combine_body(x_ref, msg_ref, den_ref, sk_ref, al_ref, g_ref, b_ref,
                  w1_ref, b1_ref, w2_ref, b2_ref, o_ref):
    sel = _head_selector()
    dexp = jnp.dot(den_ref[...], sel.T, preferred_element_type=F32)
    attn = msg_ref[...] / (dexp + 1e-16) + sk_ref[...]
    x1 = x_ref[...] + al_ref[0, 0] * attn
    h2 = _ln(x1, g_ref[...], b_ref[...])
    f = jnp.dot(jax.nn.silu(jnp.dot(h2, w1_ref[...], preferred_element_type=F32) + b1_ref[...]),
                w2_ref[...], preferred_element_type=F32) + b2_ref[...]
    o_ref[...] = x1 + f


def _head_body(x4_ref, x5_ref, x6_ref, bt_ref, ew1_ref, eb1_ref, ew2_ref,
               eb2_ref, fw1_ref, fb1_ref, fw2_ref, fb2_ref,
               xm_ref, en_ref, fo_ref):
    i = pl.program_id(0)
    xm = (x4_ref[...] + x5_ref[...] + x6_ref[...]) * (1.0 / 3.0)
    xm_ref[...] = xm
    ae = jnp.dot(jax.nn.silu(jnp.dot(xm, ew1_ref[...], preferred_element_type=F32) + eb1_ref[...]),
                 ew2_ref[...], preferred_element_type=F32) + eb2_ref[...]
    gi = i * BLK + lax.broadcasted_iota(jnp.int32, (BLK, 8), 0)
    ae = jnp.where(gi < N, ae, 0.0)
    bo = (bt_ref[...] == lax.broadcasted_iota(jnp.int32, (BLK, B), 1)).astype(F32)

    @pl.when(i == 0)
    def _():
        en_ref[...] = jnp.zeros((B, 8), F32)

    en_ref[...] = en_ref[...] + jnp.dot(bo.T, ae, preferred_element_type=F32)
    fo_ref[...] = jnp.dot(jax.nn.silu(jnp.dot(xm, fw1_ref[...], preferred_element_type=F32) + fb1_ref[...]),
                          fw2_ref[...], preferred_element_type=F32) + fb2_ref[...]


def _spec(shape, im):
    return pl.BlockSpec(shape, im)


def _w(shape):
    nd = len(shape)
    return pl.BlockSpec(shape, lambda i: (0,) * nd)


def _emb_call(an2, tab, w, b, g, bb):
    return pl.pallas_call(
        _emb_body,
        grid=(NPB,),
        in_specs=[_spec((BLK, 1), lambda i: (i, 0)), _w(tab.shape), _w(w.shape),
                  _w(b.shape), _w(g.shape), _w(bb.shape)],
        out_specs=_spec((BLK, DH), lambda i: (i, 0)),
        out_shape=jax.ShapeDtypeStruct((N_PAD, DH), F32),
    )(an2, tab, w, b, g, bb)


def _edgeattr_call(ps, pd, c, w):
    return pl.pallas_call(
        _edgeattr_body,
        grid=(NBLK,),
        in_specs=[_spec((BLK, 128), lambda i: (i, 0)), _spec((BLK, 128), lambda i: (i, 0)),
                  _w(c.shape), _w(w.shape)],
        out_specs=_spec((BLK, DH), lambda i: (i, 0)),
        out_shape=jax.ShapeDtypeStruct((E_PAD, DH), F32),
    )(ps, pd, c, w)


def _dense1_call(x, g, b, wq, bq, wk, bk, wv, bv, ws, bs):
    eb = _spec((BLK, DH), lambda i: (i, 0))
    return pl.pallas_call(
        _dense1_body,
        grid=(NPB,),
        in_specs=[eb] + [_w(a.shape) for a in (g, b, wq, bq, wk, bk, wv, bv, ws, bs)],
        out_specs=[eb, _spec((BLK, 2 * DH), lambda i: (i, 0)), eb],
        out_shape=[jax.ShapeDtypeStruct((N_PAD, DH), F32),
                   jax.ShapeDtypeStruct((N_PAD, 2 * DH), F32),
                   jax.ShapeDtypeStruct((N_PAD, DH), F32)],
    )(x, g, b, wq, bq, wk, bk, wv, bv, ws, bs)


def _logits_call(q, kve, ea, we, dst2):
    eb = _spec((BLK, DH), lambda i: (i, 0))
    return pl.pallas_call(
        _logits_body,
        grid=(NBLK,),
        in_specs=[_w((N_PAD, DH)), _spec((BLK, 2 * DH), lambda i: (i, 0)), eb,
                  _w(we.shape), _spec((BLK, 1), lambda i: (i, 0))],
        out_specs=[_spec((BLK, HEADS), lambda i: (i, 0)), _w((1, HEADS))],
        out_shape=[jax.ShapeDtypeStruct((E_PAD, HEADS), F32),
                   jax.ShapeDtypeStruct((1, HEADS), F32)],
    )(q, kve, ea, we, dst2)


def _scatter_call(a, kve, ea, we, gmax, dst2):
    eb = _spec((BLK, DH), lambda i: (i, 0))
    return pl.pallas_call(
        _scatter_body,
        grid=(NBLK,),
        in_specs=[_spec((BLK, HEADS), lambda i: (i, 0)),
                  _spec((BLK, 2 * DH), lambda i: (i, 0)), eb, _w(we.shape),
                  _w((1, HEADS)), _spec((BLK, 1), lambda i: (i, 0))],
        out_specs=[_w((N_PAD, DH)), _w((N_PAD, HEADS))],
        out_shape=[jax.ShapeDtypeStruct((N_PAD, DH), F32),
                   jax.ShapeDtypeStruct((N_PAD, HEADS), F32)],
    )(a, kve, ea, we, gmax, dst2)


def _combine_call(x, msg, den, sk, al, g, b, w1, b1, w2, b2):
    eb = _spec((BLK, DH), lambda i: (i, 0))
    return pl.pallas_call(
        _combine_body,
        grid=(NPB,),
        in_specs=[eb, eb, _spec((BLK, HEADS), lambda i: (i, 0)), eb,
                  _w(al.shape), _w(g.shape), _w(b.shape), _w(w1.shape),
                  _w(b1.shape), _w(w2.shape), _w(b2.shape)],
        out_specs=eb,
        out_shape=jax.ShapeDtypeStruct((N_PAD, DH), F32),
    )(x, msg, den, sk, al, g, b, w1, b1, w2, b2)


def _head_call(x4, x5, x6, bt2, ew1, eb1, ew2, eb2, fw1, fb1, fw2, fb2):
    eb = _spec((BLK, DH), lambda i: (i, 0))
    return pl.pallas_call(
        _head_body,
        grid=(NPB,),
        in_specs=[eb, eb, eb, _spec((BLK, 1), lambda i: (i, 0))]
                 + [_w(a.shape) for a in (ew1, eb1, ew2, eb2, fw1, fb1, fw2, fb2)],
        out_specs=[eb, _w((B, 8)), _spec((BLK, 8), lambda i: (i, 0))],
        out_shape=[jax.ShapeDtypeStruct((N_PAD, DH), F32),
                   jax.ShapeDtypeStruct((B, 8), F32),
                   jax.ShapeDtypeStruct((N_PAD, 8), F32)],
    )(x4, x5, x6, bt2, ew1, eb1, ew2, eb2, fw1, fb1, fw2, fb2)


CHUNK = 64


def _sc_gather2(table, idx_a, idx_b, d):
    info = plsc.get_sparse_core_info()
    nc, ns = info.num_cores, info.num_subcores
    nw = nc * ns
    bpw = E_PAD // nw
    nch = bpw // CHUNK
    mesh = plsc.VectorSubcoreMesh(core_axis_name="c", subcore_axis_name="s")

    @functools.partial(
        pl.kernel, mesh=mesh,
        out_type=[jax.ShapeDtypeStruct((E_PAD, d), F32)] * 2,
        scratch_types=[
            pltpu.VMEM((CHUNK,), jnp.int32), pltpu.VMEM((CHUNK,), jnp.int32),
            pltpu.VMEM((CHUNK, d), F32), pltpu.VMEM((CHUNK, d), F32),
            pltpu.SemaphoreType.DMA, pltpu.SemaphoreType.DMA,
        ],
    )
    def k(tab_h, ia_h, ib_h, oa_h, ob_h, ia_v, ib_v, ra_v, rb_v, sa, sb):
        wid = lax.axis_index("s") * nc + lax.axis_index("c")
        base = wid * bpw

        def body(c, _):
            off = base + c * CHUNK
            pltpu.sync_copy(ia_h.at[pl.ds(off, CHUNK)], ia_v)
            pltpu.sync_copy(ib_h.at[pl.ds(off, CHUNK)], ib_v)
            ca = pltpu.async_copy(tab_h.at[ia_v], ra_v, sa)
            cb = pltpu.async_copy(tab_h.at[ib_v], rb_v, sb)
            ca.wait()
            cb.wait()
            pltpu.sync_copy(ra_v, oa_h.at[pl.ds(off, CHUNK)])
            pltpu.sync_copy(rb_v, ob_h.at[pl.ds(off, CHUNK)])
            return _

        lax.fori_loop(0, nch, body, None)

    return k(table, idx_a, idx_b)


def _sc_gather1(table, idx, d):
    info = plsc.get_sparse_core_info()
    nc, ns = info.num_cores, info.num_subcores
    nw = nc * ns
    bpw = E_PAD // nw
    nch = bpw // CHUNK
    mesh = plsc.VectorSubcoreMesh(core_axis_name="c", subcore_axis_name="s")

    @functools.partial(
        pl.kernel, mesh=mesh,
        out_type=jax.ShapeDtypeStruct((E_PAD, d), F32),
        scratch_types=[
            pltpu.VMEM((CHUNK,), jnp.int32),
            pltpu.VMEM((CHUNK, d), F32),
            pltpu.SemaphoreType.DMA,
        ],
    )
    def k(tab_h, i_h, o_h, i_v, r_v, sem):
        wid = lax.axis_index("s") * nc + lax.axis_index("c")
        base = wid * bpw

        def body(c, _):
            off = base + c * CHUNK
            pltpu.sync_copy(i_h.at[pl.ds(off, CHUNK)], i_v)
            pltpu.async_copy(tab_h.at[i_v], r_v, sem).wait()
            pltpu.sync_copy(r_v, o_h.at[pl.ds(off, CHUNK)])
            return _

        lax.fori_loop(0, nch, body, None)

    return k(table, idx)


def kernel(atomic_numbers, pos, edge_index, batch, params):
    p = params
    r2 = lambda a: a.reshape(1, -1).astype(F32)

    src = edge_index[0].astype(jnp.int32)
    dst = edge_index[1].astype(jnp.int32)
    perm = jnp.argsort(dst)
    src_s = src[perm]
    dst_s = dst[perm]
    padn = E_PAD - E
    src_sp = jnp.concatenate([src_s, jnp.zeros((padn,), jnp.int32)])
    dst_gp = jnp.concatenate([dst_s, jnp.zeros((padn,), jnp.int32)])
    dst_wp = jnp.concatenate([dst_s, jnp.full((padn,), dst_s[-1], jnp.int32)])
    dst2 = dst_wp.reshape(E_PAD, 1)
    an2 = jnp.concatenate([atomic_numbers.astype(jnp.int32),
                           jnp.zeros((N_PAD - N,), jnp.int32)]).reshape(N_PAD, 1)
    bt2 = jnp.concatenate([batch.astype(jnp.int32),
                           jnp.full((N_PAD - N,), B - 1, jnp.int32)]).reshape(N_PAD, 1)
    posp = jnp.pad(pos.astype(F32), ((0, 0), (0, 125)))

    tab = jnp.concatenate([p['elem_emb'], p['radius_emb'], p['en_emb'], p['ie_emb']],
                          axis=1).astype(F32)
    x = _emb_call(an2, tab, p['proj_W'].astype(F32), r2(p['proj_b']),
                  r2(p['proj_ln_g']), r2(p['proj_ln_b']))

    ps, pd = _sc_gather2(posp, src_sp, dst_gp, 128)
    ea = _edgeattr_call(ps, pd, r2(p['rbf_centers']), r2(p['rbf_widths']))

    feats = [x]
    for l in range(L):
        q, kv, sk = _dense1_call(
            x, r2(p['n1_g'][l]), r2(p['n1_b'][l]),
            p['Wq'][l], r2(p['bq'][l]), p['Wk'][l], r2(p['bk'][l]),
            p['Wv'][l], r2(p['bv'][l]), p['Wskip'][l], r2(p['bskip'][l]))
        kve = _sc_gather1(kv, src_sp, 2 * DH)
        we = p['We'][l].astype(F32)
        a, gmax = _logits_call(q, kve, ea, we, dst2)
        msg, den = _scatter_call(a, kve, ea, we, gmax, dst2)
        x = _combine_call(x, msg, den, sk, p['alpha'][l].reshape(1, 1),
                          r2(p['n2_g'][l]), r2(p['n2_b'][l]),
                          p['f_W1'][l], r2(p['f_b1'][l]),
                          p['f_W2'][l], r2(p['f_b2'][l]))
        feats.append(x)

    ew2 = jnp.pad(p['e_W2'].astype(F32), ((0, 0), (0, 7)))
    eb2 = jnp.pad(p['e_b2'].astype(F32).reshape(1, 1), ((0, 0), (0, 7)))
    fw2 = jnp.pad(p['fr_W2'].astype(F32), ((0, 0), (0, 5)))
    fb2 = jnp.pad(p['fr_b2'].astype(F32).reshape(1, 3), ((0, 0), (0, 5)))
    xm, en, fo = _head_call(feats[L - 2], feats[L - 1], feats[L], bt2,
                            p['e_W1'].astype(F32), r2(p['e_b1']), ew2, eb2,
                            p['fr_W1'].astype(F32), r2(p['fr_b1']), fw2, fb2)
    energy = en[:, 0]
    forces = fo[:N, :3]
    return energy, forces, xm[:N]


# speedup vs baseline: 13.8876x; 1.0263x over previous
import functools

import jax
import jax.numpy as jnp
from jax import lax
from jax.experimental import pallas as pl
from jax.experimental.pallas import tpu as pltpu
from jax.experimental.pallas import tpu_sc as plsc

N, E, L, DH, HEADS, HD, B = 10000, 160000, 6, 256, 8, 32, 64
BLK = 512
NBLK = 320
E_PAD = NBLK * BLK
NPB = 21
N_PAD = NPB * BLK
WIN = 528
F32 = jnp.float32


def _ln(x, g, b):
    mu = jnp.mean(x, axis=-1, keepdims=True)
    var = jnp.mean((x - mu) ** 2, axis=-1, keepdims=True)
    return (x - mu) * jax.lax.rsqrt(var + 1e-5) * g + b


def _head_selector():
    c = lax.broadcasted_iota(jnp.int32, (DH, HEADS), 0)
    h = lax.broadcasted_iota(jnp.int32, (DH, HEADS), 1)
    return (c // HD == h).astype(F32)


def _emb_body(an_ref, tab_ref, w_ref, b_ref, g_ref, bb_ref, o_ref):
    an = an_ref[...]
    oh = (an == lax.broadcasted_iota(jnp.int32, (BLK, 119), 1)).astype(F32)
    t = jnp.dot(oh, tab_ref[...], preferred_element_type=F32)
    y = jnp.dot(t, w_ref[...], preferred_element_type=F32) + b_ref[...]
    o_ref[...] = jax.nn.silu(_ln(y, g_ref[...], bb_ref[...]))


def _edgeattr_body(ps_ref, pd_ref, c_ref, w_ref, o_ref):
    diff = ps_ref[...] - pd_ref[...]
    d2 = jnp.sum(diff * diff, axis=1, keepdims=True)
    d = jnp.sqrt(d2)
    gamma = 1.0 / (2.0 * w_ref[...] ** 2)
    rbf = jnp.exp(-gamma * (d - c_ref[...]) ** 2)
    cut = 0.5 * (jnp.cos(jnp.pi * d / 10.0) + 1.0) * (d < 10.0).astype(F32)
    o_ref[...] = rbf * cut


def _dense1_body(x_ref, g_ref, b_ref, wq_ref, bq_ref, wk_ref, bk_ref,
                 wv_ref, bv_ref, ws_ref, bs_ref,
                 q_ref, kv_ref, s_ref):
    h = _ln(x_ref[...], g_ref[...], b_ref[...])
    q_ref[...] = jnp.dot(h, wq_ref[...], preferred_element_type=F32) + bq_ref[...]
    kv_ref[:, :DH] = jnp.dot(h, wk_ref[...], preferred_element_type=F32) + bk_ref[...]
    kv_ref[:, DH:] = jnp.dot(h, wv_ref[...], preferred_element_type=F32) + bv_ref[...]
    s_ref[...] = jnp.dot(h, ws_ref[...], preferred_element_type=F32) + bs_ref[...]


def _logits_body(q_ref, kve_ref, ea_ref, we_ref, dst_ref, a_ref, gmax_ref):
    i = pl.program_id(0)
    dst = dst_ref[...]
    start = (dst_ref[0, 0] // 8) * 8
    li = dst - start
    oh = (li == lax.broadcasted_iota(jnp.int32, (BLK, WIN), 1)).astype(F32)
    qe = jnp.dot(oh, q_ref[pl.ds(start, WIN), :], preferred_element_type=F32)
    e_blk = jnp.dot(ea_ref[...], we_ref[...], preferred_element_type=F32)
    prod = qe * (kve_ref[:, :DH] + e_blk)
    a = jnp.dot(prod, _head_selector(), preferred_element_type=F32)
    a = a * (1.0 / (HD ** 0.5))
    gi = i * BLK + lax.broadcasted_iota(jnp.int32, (BLK, HEADS), 0)
    a = jnp.where(gi < E, a, -1e30)
    a_ref[...] = a

    @pl.when(i == 0)
    def _():
        gmax_ref[...] = jnp.full((1, HEADS), -1e30, F32)

    gmax_ref[...] = jnp.maximum(gmax_ref[...], jnp.max(a, axis=0, keepdims=True))


def _scatter_body(a_ref, kve_ref, ea_ref, we_ref, gmax_ref, dst_ref,
                  msg_ref, den_ref):
    i = pl.program_id(0)

    @pl.when(i == 0)
    def _():
        msg_ref[...] = jnp.zeros((N_PAD, DH), F32)
        den_ref[...] = jnp.zeros((N_PAD, HEADS), F32)

    dst = dst_ref[...]
    start = (dst_ref[0, 0] // 8) * 8
    li = dst - start
    oh = (li == lax.broadcasted_iota(jnp.int32, (BLK, WIN), 1)).astype(F32)
    w = jnp.exp(a_ref[...] - gmax_ref[...])
    sel = _head_selector()
    e_blk = jnp.dot(ea_ref[...], we_ref[...], preferred_element_type=F32)
    vpe = kve_ref[:, DH:] + e_blk
    wb = jnp.dot(w, sel.T, preferred_element_type=F32)
    acc = jnp.dot(oh.T, vpe * wb, preferred_element_type=F32)
    dacc = jnp.dot(oh.T, w, preferred_element_type=F32)
    msg_ref[pl.ds(start, WIN), :] = msg_ref[pl.ds(start, WIN), :] + acc
    den_ref[pl.ds(start, WIN), :] = den_ref[pl.ds(start, WIN), :] + dacc


def _combine_body(x_ref, msg_ref, den_ref, sk_ref, al_ref, g_ref, b_ref,
                  w1_ref, b1_ref, w2_ref, b2_ref, o_ref):
    sel = _head_selector()
    dexp = jnp.dot(den_ref[...], sel.T, preferred_element_type=F32)
    attn = msg_ref[...] / (dexp + 1e-16) + sk_ref[...]
    x1 = x_ref[...] + al_ref[0, 0] * attn
    h2 = _ln(x1, g_ref[...], b_ref[...])
    f = jnp.dot(jax.nn.silu(jnp.dot(h2, w1_ref[...], preferred_element_type=F32) + b1_ref[...]),
                w2_ref[...], preferred_element_type=F32) + b2_ref[...]
    o_ref[...] = x1 + f


def _head_body(x4_ref, x5_ref, x6_ref, bt_ref, ew1_ref, eb1_ref, ew2_ref,
               eb2_ref, fw1_ref, fb1_ref, fw2_ref, fb2_ref,
               xm_ref, en_ref, fo_ref):
    i = pl.program_id(0)
    xm = (x4_ref[...] + x5_ref[...] + x6_ref[...]) * (1.0 / 3.0)
    xm_ref[...] = xm
    ae = jnp.dot(jax.nn.silu(jnp.dot(xm, ew1_ref[...], preferred_element_type=F32) + eb1_ref[...]),
                 ew2_ref[...], preferred_element_type=F32) + eb2_ref[...]
    gi = i * BLK + lax.broadcasted_iota(jnp.int32, (BLK, 8), 0)
    ae = jnp.where(gi < N, ae, 0.0)
    bo = (bt_ref[...] == lax.broadcasted_iota(jnp.int32, (BLK, B), 1)).astype(F32)

    @pl.when(i == 0)
    def _():
        en_ref[...] = jnp.zeros((B, 8), F32)

    en_ref[...] = en_ref[...] + jnp.dot(bo.T, ae, preferred_element_type=F32)
    fo_ref[...] = jnp.dot(jax.nn.silu(jnp.dot(xm, fw1_ref[...], preferred_element_type=F32) + fb1_ref[...]),
                          fw2_ref[...], preferred_element_type=F32) + fb2_ref[...]


def _spec(shape, im):
    return pl.BlockSpec(shape, im)


def _w(shape):
    nd = len(shape)
    return pl.BlockSpec(shape, lambda i: (0,) * nd)


def _emb_call(an2, tab, w, b, g, bb):
    return pl.pallas_call(
        _emb_body,
        grid=(NPB,),
        in_specs=[_spec((BLK, 1), lambda i: (i, 0)), _w(tab.shape), _w(w.shape),
                  _w(b.shape), _w(g.shape), _w(bb.shape)],
        out_specs=_spec((BLK, DH), lambda i: (i, 0)),
        out_shape=jax.ShapeDtypeStruct((N_PAD, DH), F32),
    )(an2, tab, w, b, g, bb)


def _edgeattr_call(ps, pd, c, w):
    return pl.pallas_call(
        _edgeattr_body,
        grid=(NBLK,),
        in_specs=[_spec((BLK, 128), lambda i: (i, 0)), _spec((BLK, 128), lambda i: (i, 0)),
                  _w(c.shape), _w(w.shape)],
        out_specs=_spec((BLK, DH), lambda i: (i, 0)),
        out_shape=jax.ShapeDtypeStruct((E_PAD, DH), F32),
    )(ps, pd, c, w)


def _dense1_call(x, g, b, wq, bq, wk, bk, wv, bv, ws, bs):
    eb = _spec((BLK, DH), lambda i: (i, 0))
    return pl.pallas_call(
        _dense1_body,
        grid=(NPB,),
        in_specs=[eb] + [_w(a.shape) for a in (g, b, wq, bq, wk, bk, wv, bv, ws, bs)],
        out_specs=[eb, _spec((BLK, 2 * DH), lambda i: (i, 0)), eb],
        out_shape=[jax.ShapeDtypeStruct((N_PAD, DH), F32),
                   jax.ShapeDtypeStruct((N_PAD, 2 * DH), F32),
                   jax.ShapeDtypeStruct((N_PAD, DH), F32)],
    )(x, g, b, wq, bq, wk, bk, wv, bv, ws, bs)


def _logits_call(q, kve, ea, we, dst2):
    eb = _spec((BLK, DH), lambda i: (i, 0))
    return pl.pallas_call(
        _logits_body,
        grid=(NBLK,),
        in_specs=[_w((N_PAD, DH)), _spec((BLK, 2 * DH), lambda i: (i, 0)), eb,
                  _w(we.shape), _spec((BLK, 1), lambda i: (i, 0))],
        out_specs=[_spec((BLK, HEADS), lambda i: (i, 0)), _w((1, HEADS))],
        out_shape=[jax.ShapeDtypeStruct((E_PAD, HEADS), F32),
                   jax.ShapeDtypeStruct((1, HEADS), F32)],
    )(q, kve, ea, we, dst2)


def _scatter_call(a, kve, ea, we, gmax, dst2):
    eb = _spec((BLK, DH), lambda i: (i, 0))
    return pl.pallas_call(
        _scatter_body,
        grid=(NBLK,),
        in_specs=[_spec((BLK, HEADS), lambda i: (i, 0)),
                  _spec((BLK, 2 * DH), lambda i: (i, 0)), eb, _w(we.shape),
                  _w((1, HEADS)), _spec((BLK, 1), lambda i: (i, 0))],
        out_specs=[_w((N_PAD, DH)), _w((N_PAD, HEADS))],
        out_shape=[jax.ShapeDtypeStruct((N_PAD, DH), F32),
                   jax.ShapeDtypeStruct((N_PAD, HEADS), F32)],
    )(a, kve, ea, we, gmax, dst2)


def _combine_call(x, msg, den, sk, al, g, b, w1, b1, w2, b2):
    eb = _spec((BLK, DH), lambda i: (i, 0))
    return pl.pallas_call(
        _combine_body,
        grid=(NPB,),
        in_specs=[eb, eb, _spec((BLK, HEADS), lambda i: (i, 0)), eb,
                  _w(al.shape), _w(g.shape), _w(b.shape), _w(w1.shape),
                  _w(b1.shape), _w(w2.shape), _w(b2.shape)],
        out_specs=eb,
        out_shape=jax.ShapeDtypeStruct((N_PAD, DH), F32),
    )(x, msg, den, sk, al, g, b, w1, b1, w2, b2)


def _head_call(x4, x5, x6, bt2, ew1, eb1, ew2, eb2, fw1, fb1, fw2, fb2):
    eb = _spec((BLK, DH), lambda i: (i, 0))
    return pl.pallas_call(
        _head_body,
        grid=(NPB,),
        in_specs=[eb, eb, eb, _spec((BLK, 1), lambda i: (i, 0))]
                 + [_w(a.shape) for a in (ew1, eb1, ew2, eb2, fw1, fb1, fw2, fb2)],
        out_specs=[eb, _w((B, 8)), _spec((BLK, 8), lambda i: (i, 0))],
        out_shape=[jax.ShapeDtypeStruct((N_PAD, DH), F32),
                   jax.ShapeDtypeStruct((B, 8), F32),
                   jax.ShapeDtypeStruct((N_PAD, 8), F32)],
    )(x4, x5, x6, bt2, ew1, eb1, ew2, eb2, fw1, fb1, fw2, fb2)


CHUNK = 128


def _sc_gather2(table, idx_a, idx_b, d):
    info = plsc.get_sparse_core_info()
    nc, ns = info.num_cores, info.num_subcores
    nw = nc * ns
    bpw = E_PAD // nw
    nch = bpw // CHUNK
    mesh = plsc.VectorSubcoreMesh(core_axis_name="c", subcore_axis_name="s")

    @functools.partial(
        pl.kernel, mesh=mesh,
        out_type=[jax.ShapeDtypeStruct((E_PAD, d), F32)] * 2,
        scratch_types=[
            pltpu.VMEM((CHUNK,), jnp.int32), pltpu.VMEM((CHUNK,), jnp.int32),
            pltpu.VMEM((CHUNK, d), F32), pltpu.VMEM((CHUNK, d), F32),
            pltpu.SemaphoreType.DMA, pltpu.SemaphoreType.DMA,
        ],
    )
    def k(tab_h, ia_h, ib_h, oa_h, ob_h, ia_v, ib_v, ra_v, rb_v, sa, sb):
        wid = lax.axis_index("s") * nc + lax.axis_index("c")
        base = wid * bpw

        def body(c, _):
            off = base + c * CHUNK
            pltpu.sync_copy(ia_h.at[pl.ds(off, CHUNK)], ia_v)
            pltpu.sync_copy(ib_h.at[pl.ds(off, CHUNK)], ib_v)
            ca = pltpu.async_copy(tab_h.at[ia_v], ra_v, sa)
            cb = pltpu.async_copy(tab_h.at[ib_v], rb_v, sb)
            ca.wait()
            cb.wait()
            pltpu.sync_copy(ra_v, oa_h.at[pl.ds(off, CHUNK)])
            pltpu.sync_copy(rb_v, ob_h.at[pl.ds(off, CHUNK)])
            return _

        lax.fori_loop(0, nch, body, None)

    return k(table, idx_a, idx_b)


def _sc_gather1(table, idx, d):
    info = plsc.get_sparse_core_info()
    nc, ns = info.num_cores, info.num_subcores
    nw = nc * ns
    bpw = E_PAD // nw
    nch = bpw // CHUNK
    mesh = plsc.VectorSubcoreMesh(core_axis_name="c", subcore_axis_name="s")

    @functools.partial(
        pl.kernel, mesh=mesh,
        out_type=jax.ShapeDtypeStruct((E_PAD, d), F32),
        scratch_types=[
            pltpu.VMEM((CHUNK,), jnp.int32),
            pltpu.VMEM((CHUNK, d), F32),
            pltpu.SemaphoreType.DMA,
        ],
    )
    def k(tab_h, i_h, o_h, i_v, r_v, sem):
        wid = lax.axis_index("s") * nc + lax.axis_index("c")
        base = wid * bpw

        def body(c, _):
            off = base + c * CHUNK
            pltpu.sync_copy(i_h.at[pl.ds(off, CHUNK)], i_v)
            pltpu.async_copy(tab_h.at[i_v], r_v, sem).wait()
            pltpu.sync_copy(r_v, o_h.at[pl.ds(off, CHUNK)])
            return _

        lax.fori_loop(0, nch, body, None)

    return k(table, idx)


def kernel(atomic_numbers, pos, edge_index, batch, params):
    p = params
    r2 = lambda a: a.reshape(1, -1).astype(F32)

    src = edge_index[0].astype(jnp.int32)
    dst = edge_index[1].astype(jnp.int32)
    perm = jnp.argsort(dst)
    src_s = src[perm]
    dst_s = dst[perm]
    padn = E_PAD - E
    src_sp = jnp.concatenate([src_s, jnp.zeros((padn,), jnp.int32)])
    dst_gp = jnp.concatenate([dst_s, jnp.zeros((padn,), jnp.int32)])
    dst_wp = jnp.concatenate([dst_s, jnp.full((padn,), dst_s[-1], jnp.int32)])
    dst2 = dst_wp.reshape(E_PAD, 1)
    an2 = jnp.concatenate([atomic_numbers.astype(jnp.int32),
                           jnp.zeros((N_PAD - N,), jnp.int32)]).reshape(N_PAD, 1)
    bt2 = jnp.concatenate([batch.astype(jnp.int32),
                           jnp.full((N_PAD - N,), B - 1, jnp.int32)]).reshape(N_PAD, 1)
    posp = jnp.pad(pos.astype(F32), ((0, 0), (0, 125)))

    tab = jnp.concatenate([p['elem_emb'], p['radius_emb'], p['en_emb'], p['ie_emb']],
                          axis=1).astype(F32)
    x = _emb_call(an2, tab, p['proj_W'].astype(F32), r2(p['proj_b']),
                  r2(p['proj_ln_g']), r2(p['proj_ln_b']))

    ps, pd = _sc_gather2(posp, src_sp, dst_gp, 128)
    ea = _edgeattr_call(ps, pd, r2(p['rbf_centers']), r2(p['rbf_widths']))

    feats = [x]
    for l in range(L):
        q, kv, sk = _dense1_call(
            x, r2(p['n1_g'][l]), r2(p['n1_b'][l]),
            p['Wq'][l], r2(p['bq'][l]), p['Wk'][l], r2(p['bk'][l]),
            p['Wv'][l], r2(p['bv'][l]), p['Wskip'][l], r2(p['bskip'][l]))
        kve = _sc_gather1(kv, src_sp, 2 * DH)
        we = p['We'][l].astype(F32)
        a, gmax = _logits_call(q, kve, ea, we, dst2)
        msg, den = _scatter_call(a, kve, ea, we, gmax, dst2)
        x = _combine_call(x, msg, den, sk, p['alpha'][l].reshape(1, 1),
                          r2(p['n2_g'][l]), r2(p['n2_b'][l]),
                          p['f_W1'][l], r2(p['f_b1'][l]),
                          p['f_W2'][l], r2(p['f_b2'][l]))
        feats.append(x)

    ew2 = jnp.pad(p['e_W2'].astype(F32), ((0, 0), (0, 7)))
    eb2 = jnp.pad(p['e_b2'].astype(F32).reshape(1, 1), ((0, 0), (0, 7)))
    fw2 = jnp.pad(p['fr_W2'].astype(F32), ((0, 0), (0, 5)))
    fb2 = jnp.pad(p['fr_b2'].astype(F32).reshape(1, 3), ((0, 0), (0, 5)))
    xm, en, fo = _head_call(feats[L - 2], feats[L - 1], feats[L], bt2,
                            p['e_W1'].astype(F32), r2(p['e_b1']), ew2, eb2,
                            p['fr_W1'].astype(F32), r2(p['fr_b1']), fw2, fb2)
    energy = en[:, 0]
    forces = fo[:N, :3]
    return energy, forces, xm[:N]
